# baseline (speedup 1.0000x reference)
import math

import jax
import jax.numpy as jnp
from jax.experimental import pallas as pl
from jax.experimental.pallas import tpu as pltpu

D0, H, W = 21, 80, 64
_INV = 1.0 / math.sqrt(1.0 + 1e-3)

_LAYER_PLAN = [
    ('subm', 1, ((1, 1), (1, 1), (1, 1)), 4),
    ('subm', 1, ((1, 1), (1, 1), (1, 1)), 4),
    ('sp',   2, ((1, 1), (1, 1), (1, 1)), 2),
    ('subm', 1, ((1, 1), (1, 1), (1, 1)), 2),
    ('subm', 1, ((1, 1), (1, 1), (1, 1)), 2),
    ('subm', 1, ((1, 1), (1, 1), (1, 1)), 2),
    ('sp',   2, ((0, 0), (1, 1), (1, 1)), 2),
    ('subm', 1, ((1, 1), (1, 1), (1, 1)), 2),
    ('subm', 1, ((1, 1), (1, 1), (1, 1)), 2),
    ('subm', 1, ((1, 1), (1, 1), (1, 1)), 2),
    ('sp',  2, ((0, 0), (0, 0), (0, 0)), 2),
]


def _make_body(kind, ky, kx, cin, cout, sz, p):
    nref = sz * (p - 1) + 3

    def body(*refs):
        xs = refs[:nref]
        if kind == 'subm':
            w_ref, g_ref, b_ref, m_ref = refs[nref:nref + 4]
            y_ref = refs[-1]
        else:
            ms = refs[nref:2 * nref]
            w_ref, g_ref, b_ref = refs[2 * nref:2 * nref + 3]
            y_ref, mo_ref = refs[-2], refs[-1]

        acc = None
        for dz in range(3):
            if p == 1:
                pk = xs[dz][0]
            else:
                pk = jnp.concatenate([xs[j * sz + dz][0] for j in range(p)],
                                     axis=-1)
            wins = [pk[dy:dy + H, dx:dx + W, :]
                    for dy in range(ky) for dx in range(kx)]
            cat = wins[0] if len(wins) == 1 else jnp.concatenate(wins, axis=-1)
            d = jax.lax.dot_general(cat, w_ref[dz], (((2,), (0,)), ((), ())),
                                    preferred_element_type=jnp.float32)
            acc = d if acc is None else acc + d

        y = acc * (_INV * g_ref[0]) + b_ref[0]
        if kind == 'subm':
            if p == 1:
                m = m_ref[0].astype(jnp.float32)[:, :, None]
            else:
                m = jnp.concatenate(
                    [jnp.broadcast_to(m_ref[j].astype(jnp.float32)[:, :, None],
                                      (H, W, cout)) for j in range(p)], axis=-1)
            y = jnp.maximum(y, 0.0) * m
            for j in range(p):
                y_ref[j] = y[:, :, j * cout:(j + 1) * cout].astype(jnp.bfloat16)
        else:
            newms = []
            for j in range(p):
                msum = jnp.zeros((H, W), jnp.float32)
                for dz in range(3):
                    mb = ms[j * sz + dz]
                    for dy in range(ky):
                        for dx in range(kx):
                            msum += mb[0, dy:dy + H, dx:dx + W].astype(jnp.float32)
                newm = (msum > 0.0).astype(jnp.float32)
                newms.append(newm)
                mo_ref[j] = newm.astype(jnp.bfloat16)
            if p == 1:
                m = newms[0][:, :, None]
            else:
                m = jnp.concatenate(
                    [jnp.broadcast_to(nm[:, :, None], (H, W, cout))
                     for nm in newms], axis=-1)
            y = jnp.maximum(y, 0.0) * m
            for j in range(p):
                y_ref[j] = y[:, :, j * cout:(j + 1) * cout].astype(jnp.bfloat16)
    return body


def _conv_layer(x, mask, w, gamma, beta, kind, sz, pad, p):
    kz, ky, kx, cin, cout = w.shape
    pz, py, px = pad
    din = x.shape[0]
    dout = (din + pz[0] + pz[1] - kz) // sz + 1
    g = -(-dout // p)
    need = (g * p - 1) * sz + kz
    extra = need - (din + pz[0] + pz[1])
    hp, wp = H + py[0] + py[1], W + px[0] + px[1]
    xp = jnp.pad(x, ((pz[0], pz[1] + extra), (py[0], py[1]),
                     (px[0], px[1]), (0, 0)))
    wr = w.reshape(kz, ky * kx, cin, cout)
    eye = jnp.eye(p, dtype=w.dtype)
    wf = jnp.einsum('ztco,sj->ztscjo', wr, eye).reshape(
        kz, ky * kx * p * cin, p * cout).astype(jnp.bfloat16)
    g2 = jnp.tile(gamma, p).reshape(1, p * cout)
    b2 = jnp.tile(beta, p).reshape(1, p * cout)
    nref = sz * (p - 1) + 3

    def slab(k):
        return pl.BlockSpec((1, hp, wp, cin),
                            lambda d, k=k: (d * p * sz + k, 0, 0, 0))

    def mslab(k):
        return pl.BlockSpec((1, hp, wp), lambda d, k=k: (d * p * sz + k, 0, 0))

    wspec = pl.BlockSpec((kz, ky * kx * p * cin, p * cout), lambda d: (0, 0, 0))
    vspec = pl.BlockSpec((1, p * cout), lambda d: (0, 0))
    yspec = pl.BlockSpec((p, H, W, cout), lambda d: (d, 0, 0, 0))
    mospec = pl.BlockSpec((p, H, W), lambda d: (d, 0, 0))

    cparams = pltpu.CompilerParams(dimension_semantics=("parallel",))
    body = _make_body(kind, ky, kx, cin, cout, sz, p)
    if kind == 'subm':
        mpad = jnp.pad(mask, ((0, g * p - dout), (0, 0), (0, 0)))
        y = pl.pallas_call(
            body,
            grid=(g,),
            in_specs=[slab(k) for k in range(nref)] + [wspec, vspec, vspec, mospec],
            out_specs=yspec,
            out_shape=jax.ShapeDtypeStruct((g * p, H, W, cout), jnp.bfloat16),
            compiler_params=cparams,
        )(*([xp] * nref), wf, g2, b2, mpad)
        return y[:dout], mask
    else:
        mp = jnp.pad(mask, ((pz[0], pz[1] + extra), (py[0], py[1]),
                            (px[0], px[1])))
        y, newm = pl.pallas_call(
            body,
            grid=(g,),
            in_specs=([slab(k) for k in range(nref)]
                      + [mslab(k) for k in range(nref)]
                      + [wspec, vspec, vspec]),
            out_specs=[yspec, mospec],
            out_shape=[jax.ShapeDtypeStruct((g * p, H, W, cout), jnp.bfloat16),
                       jax.ShapeDtypeStruct((g * p, H, W), jnp.bfloat16)],
            compiler_params=cparams,
        )(*([xp] * nref), *([mp] * nref), wf, g2, b2)
        return y[:dout], newm[:dout]


def kernel(voxel_features, coors, batch_size, weights, gammas, betas):
    del batch_size
    nvox, cin = voxel_features.shape
    lin = coors[:, 1] * (H * W) + coors[:, 2] * W + coors[:, 3]
    x = jnp.zeros((D0 * H * W, cin), jnp.bfloat16).at[lin].set(
        voxel_features.astype(jnp.bfloat16))
    x = x.reshape(D0, H, W, cin)
    mask = jnp.zeros((D0 * H * W,), jnp.bfloat16).at[lin].set(1.0)
    mask = mask.reshape(D0, H, W)

    for (kind, sz, pad, p), w, g, b in zip(_LAYER_PLAN, weights, gammas, betas):
        x, mask = _conv_layer(x, mask, w, g, b, kind, sz, pad, p)

    dd, _, _, c = x.shape
    out = jnp.transpose(x.astype(jnp.float32), (3, 0, 1, 2)).reshape(1, c * dd, H, W)
    return out


# speedup vs baseline: 1.0894x; 1.0894x over previous
import math

import jax
import jax.numpy as jnp
from jax.experimental import pallas as pl
from jax.experimental.pallas import tpu as pltpu

D0, H, W = 21, 80, 64
_INV = 1.0 / math.sqrt(1.0 + 1e-3)

_LAYER_PLAN = [
    ('subm', 1, ((1, 1), (1, 1), (1, 1)), 4),
    ('subm', 1, ((1, 1), (1, 1), (1, 1)), 4),
    ('sp',   2, ((1, 1), (1, 1), (1, 1)), 2),
    ('subm', 1, ((1, 1), (1, 1), (1, 1)), 1),
    ('subm', 1, ((1, 1), (1, 1), (1, 1)), 1),
    ('subm', 1, ((1, 1), (1, 1), (1, 1)), 1),
    ('sp',   2, ((0, 0), (1, 1), (1, 1)), 1),
    ('subm', 1, ((1, 1), (1, 1), (1, 1)), 1),
    ('subm', 1, ((1, 1), (1, 1), (1, 1)), 1),
    ('subm', 1, ((1, 1), (1, 1), (1, 1)), 1),
    ('sp',  2, ((0, 0), (0, 0), (0, 0)), 1),
]


def _make_body(kind, ky, kx, cin, cout, sz, p):
    nref = sz * (p - 1) + 3

    def body(*refs):
        xs = refs[:nref]
        if kind == 'subm':
            w_ref, g_ref, b_ref, m_ref = refs[nref:nref + 4]
            y_ref = refs[-1]
        else:
            ms = refs[nref:2 * nref]
            w_ref, g_ref, b_ref = refs[2 * nref:2 * nref + 3]
            y_ref, mo_ref = refs[-2], refs[-1]

        acc = None
        for dz in range(3):
            if p == 1:
                pk = xs[dz][0]
            else:
                pk = jnp.concatenate([xs[j * sz + dz][0] for j in range(p)],
                                     axis=-1)
            wins = [pk[dy:dy + H, dx:dx + W, :]
                    for dy in range(ky) for dx in range(kx)]
            cat = wins[0] if len(wins) == 1 else jnp.concatenate(wins, axis=-1)
            d = jax.lax.dot_general(cat, w_ref[dz], (((2,), (0,)), ((), ())),
                                    preferred_element_type=jnp.float32)
            acc = d if acc is None else acc + d

        y = acc * (_INV * g_ref[0]) + b_ref[0]
        if kind == 'subm':
            if p == 1:
                m = m_ref[0].astype(jnp.float32)[:, :, None]
            else:
                m = jnp.concatenate(
                    [jnp.broadcast_to(m_ref[j].astype(jnp.float32)[:, :, None],
                                      (H, W, cout)) for j in range(p)], axis=-1)
            y = jnp.maximum(y, 0.0) * m
            for j in range(p):
                y_ref[j] = y[:, :, j * cout:(j + 1) * cout].astype(jnp.bfloat16)
        else:
            newms = []
            for j in range(p):
                msum = jnp.zeros((H, W), jnp.float32)
                for dz in range(3):
                    mb = ms[j * sz + dz]
                    for dy in range(ky):
                        for dx in range(kx):
                            msum += mb[0, dy:dy + H, dx:dx + W].astype(jnp.float32)
                newm = (msum > 0.0).astype(jnp.float32)
                newms.append(newm)
                mo_ref[j] = newm.astype(jnp.bfloat16)
            if p == 1:
                m = newms[0][:, :, None]
            else:
                m = jnp.concatenate(
                    [jnp.broadcast_to(nm[:, :, None], (H, W, cout))
                     for nm in newms], axis=-1)
            y = jnp.maximum(y, 0.0) * m
            for j in range(p):
                y_ref[j] = y[:, :, j * cout:(j + 1) * cout].astype(jnp.bfloat16)
    return body


def _conv_layer(x, mask, w, gamma, beta, kind, sz, pad, p):
    kz, ky, kx, cin, cout = w.shape
    pz, py, px = pad
    din = x.shape[0]
    dout = (din + pz[0] + pz[1] - kz) // sz + 1
    g = -(-dout // p)
    need = (g * p - 1) * sz + kz
    extra = need - (din + pz[0] + pz[1])
    hp, wp = H + py[0] + py[1], W + px[0] + px[1]
    xp = jnp.pad(x, ((pz[0], pz[1] + extra), (py[0], py[1]),
                     (px[0], px[1]), (0, 0)))
    wr = w.reshape(kz, ky * kx, cin, cout)
    eye = jnp.eye(p, dtype=w.dtype)
    wf = jnp.einsum('ztco,sj->ztscjo', wr, eye).reshape(
        kz, ky * kx * p * cin, p * cout).astype(jnp.bfloat16)
    g2 = jnp.tile(gamma, p).reshape(1, p * cout)
    b2 = jnp.tile(beta, p).reshape(1, p * cout)
    nref = sz * (p - 1) + 3

    def slab(k):
        return pl.BlockSpec((1, hp, wp, cin),
                            lambda d, k=k: (d * p * sz + k, 0, 0, 0))

    def mslab(k):
        return pl.BlockSpec((1, hp, wp), lambda d, k=k: (d * p * sz + k, 0, 0))

    wspec = pl.BlockSpec((kz, ky * kx * p * cin, p * cout), lambda d: (0, 0, 0))
    vspec = pl.BlockSpec((1, p * cout), lambda d: (0, 0))
    yspec = pl.BlockSpec((p, H, W, cout), lambda d: (d, 0, 0, 0))
    mospec = pl.BlockSpec((p, H, W), lambda d: (d, 0, 0))

    cparams = pltpu.CompilerParams(dimension_semantics=("parallel",))
    body = _make_body(kind, ky, kx, cin, cout, sz, p)
    if kind == 'subm':
        mpad = jnp.pad(mask, ((0, g * p - dout), (0, 0), (0, 0)))
        y = pl.pallas_call(
            body,
            grid=(g,),
            in_specs=[slab(k) for k in range(nref)] + [wspec, vspec, vspec, mospec],
            out_specs=yspec,
            out_shape=jax.ShapeDtypeStruct((g * p, H, W, cout), jnp.bfloat16),
            compiler_params=cparams,
        )(*([xp] * nref), wf, g2, b2, mpad)
        return y[:dout], mask
    else:
        mp = jnp.pad(mask, ((pz[0], pz[1] + extra), (py[0], py[1]),
                            (px[0], px[1])))
        y, newm = pl.pallas_call(
            body,
            grid=(g,),
            in_specs=([slab(k) for k in range(nref)]
                      + [mslab(k) for k in range(nref)]
                      + [wspec, vspec, vspec]),
            out_specs=[yspec, mospec],
            out_shape=[jax.ShapeDtypeStruct((g * p, H, W, cout), jnp.bfloat16),
                       jax.ShapeDtypeStruct((g * p, H, W), jnp.bfloat16)],
            compiler_params=cparams,
        )(*([xp] * nref), *([mp] * nref), wf, g2, b2)
        return y[:dout], newm[:dout]


def kernel(voxel_features, coors, batch_size, weights, gammas, betas):
    del batch_size
    nvox, cin = voxel_features.shape
    lin = coors[:, 1] * (H * W) + coors[:, 2] * W + coors[:, 3]
    x = jnp.zeros((D0 * H * W, cin), jnp.bfloat16).at[lin].set(
        voxel_features.astype(jnp.bfloat16))
    x = x.reshape(D0, H, W, cin)
    mask = jnp.zeros((D0 * H * W,), jnp.bfloat16).at[lin].set(1.0)
    mask = mask.reshape(D0, H, W)

    for (kind, sz, pad, p), w, g, b in zip(_LAYER_PLAN, weights, gammas, betas):
        x, mask = _conv_layer(x, mask, w, g, b, kind, sz, pad, p)

    dd, _, _, c = x.shape
    out = jnp.transpose(x.astype(jnp.float32), (3, 0, 1, 2)).reshape(1, c * dd, H, W)
    return out
